# baseline (speedup 1.0000x reference)
import jax
import jax.numpy as jnp
from jax.experimental import pallas as pl
from jax.experimental.pallas import tpu as pltpu

_BF = 1000
_BI = 400


def _support_body(f_ref, w_ref, o_ref):
    o_ref[...] = jnp.dot(
        f_ref[...].astype(jnp.bfloat16),
        w_ref[...],
        preferred_element_type=jnp.float32,
    ).astype(jnp.bfloat16)


def _spmm_body(adj_ref, s_ref, o_ref):
    o_ref[...] = jnp.maximum(
        jnp.dot(
            adj_ref[...].astype(jnp.bfloat16),
            s_ref[...],
            preferred_element_type=jnp.float32,
        ),
        0.0,
    )


def kernel(features, adj, weight):
    n, d_in = features.shape
    d_out = weight.shape[1]

    support = pl.pallas_call(
        _support_body,
        grid=(n // _BF,),
        in_specs=[
            pl.BlockSpec((_BF, d_in), lambda i: (i, 0)),
            pl.BlockSpec((d_in, d_out), lambda i: (0, 0)),
        ],
        out_specs=pl.BlockSpec((_BF, d_out), lambda i: (i, 0)),
        out_shape=jax.ShapeDtypeStruct((n, d_out), jnp.bfloat16),
        compiler_params=pltpu.CompilerParams(
            dimension_semantics=("parallel",),
        ),
    )(features, weight.astype(jnp.bfloat16))

    return pl.pallas_call(
        _spmm_body,
        grid=(n // _BI,),
        in_specs=[
            pl.BlockSpec((_BI, n), lambda i: (i, 0)),
            pl.BlockSpec((n, d_out), lambda i: (0, 0)),
        ],
        out_specs=pl.BlockSpec((_BI, d_out), lambda i: (i, 0)),
        out_shape=jax.ShapeDtypeStruct((n, d_out), jnp.float32),
        compiler_params=pltpu.CompilerParams(
            dimension_semantics=("parallel",),
        ),
    )(adj, support)


# speedup vs baseline: 1.0339x; 1.0339x over previous
import jax
import jax.numpy as jnp
from jax.experimental import pallas as pl
from jax.experimental.pallas import tpu as pltpu

_BI = 400
_CS = 1000


def _fused_body(w_ref, f_hbm, adj_ref, o_ref, s_ref, f_buf, sem):
    t = pl.program_id(0)
    n_rows = f_hbm.shape[0]

    @pl.when(t == 0)
    def _support():
        n_chunks = n_rows // _CS

        def chunk_copy(j):
            return pltpu.make_async_copy(
                f_hbm.at[pl.ds(j * _CS, _CS), :], f_buf.at[j % 2], sem.at[j % 2]
            )

        chunk_copy(0).start()
        for j in range(n_chunks):
            if j + 1 < n_chunks:
                chunk_copy(j + 1).start()
            chunk_copy(j).wait()
            s_ref[j * _CS:(j + 1) * _CS, :] = jnp.dot(
                f_buf[j % 2].astype(jnp.bfloat16),
                w_ref[...],
                preferred_element_type=jnp.float32,
            ).astype(jnp.bfloat16)

    o_ref[...] = jnp.maximum(
        jnp.dot(
            adj_ref[...].astype(jnp.bfloat16),
            s_ref[...],
            preferred_element_type=jnp.float32,
        ),
        0.0,
    )


def kernel(features, adj, weight):
    n, d_in = features.shape
    d_out = weight.shape[1]

    return pl.pallas_call(
        _fused_body,
        grid=(n // _BI,),
        in_specs=[
            pl.BlockSpec((d_in, d_out), lambda i: (0, 0)),
            pl.BlockSpec(memory_space=pl.ANY),
            pl.BlockSpec((_BI, n), lambda i: (i, 0)),
        ],
        out_specs=pl.BlockSpec((_BI, d_out), lambda i: (i, 0)),
        out_shape=jax.ShapeDtypeStruct((n, d_out), jnp.float32),
        scratch_shapes=[
            pltpu.VMEM((n, d_out), jnp.bfloat16),
            pltpu.VMEM((2, _CS, d_in), jnp.float32),
            pltpu.SemaphoreType.DMA((2,)),
        ],
        compiler_params=pltpu.CompilerParams(
            dimension_semantics=("arbitrary",),
        ),
    )(weight.astype(jnp.bfloat16), features, adj)


# speedup vs baseline: 1.0451x; 1.0109x over previous
import jax
import jax.numpy as jnp
from jax.experimental import pallas as pl
from jax.experimental.pallas import tpu as pltpu

_BI = 400
_CS = 2000


def _fused_body(w_ref, f_hbm, adj_ref, o_ref, s_ref, f_buf, sem):
    t = pl.program_id(0)
    n_rows = f_hbm.shape[0]

    @pl.when(t == 0)
    def _support():
        n_chunks = n_rows // _CS

        def chunk_copy(j):
            return pltpu.make_async_copy(
                f_hbm.at[pl.ds(j * _CS, _CS), :], f_buf.at[j % 2], sem.at[j % 2]
            )

        chunk_copy(0).start()
        for j in range(n_chunks):
            if j + 1 < n_chunks:
                chunk_copy(j + 1).start()
            chunk_copy(j).wait()
            s_ref[j * _CS:(j + 1) * _CS, :] = jnp.dot(
                f_buf[j % 2].astype(jnp.bfloat16),
                w_ref[...],
                preferred_element_type=jnp.float32,
            ).astype(jnp.bfloat16)

    acc = jnp.dot(
        adj_ref[:, :5120].astype(jnp.bfloat16),
        s_ref[:5120, :],
        preferred_element_type=jnp.float32,
    )
    acc += jnp.dot(
        adj_ref[:, 5120:].astype(jnp.bfloat16),
        s_ref[5120:, :],
        preferred_element_type=jnp.float32,
    )
    o_ref[...] = jnp.maximum(acc, 0.0)


def kernel(features, adj, weight):
    n, d_in = features.shape
    d_out = weight.shape[1]

    return pl.pallas_call(
        _fused_body,
        grid=(n // _BI,),
        in_specs=[
            pl.BlockSpec((d_in, d_out), lambda i: (0, 0)),
            pl.BlockSpec(memory_space=pl.ANY),
            pl.BlockSpec((_BI, n), lambda i: (i, 0)),
        ],
        out_specs=pl.BlockSpec((_BI, d_out), lambda i: (i, 0)),
        out_shape=jax.ShapeDtypeStruct((n, d_out), jnp.float32),
        scratch_shapes=[
            pltpu.VMEM((n, d_out), jnp.bfloat16),
            pltpu.VMEM((2, _CS, d_in), jnp.float32),
            pltpu.SemaphoreType.DMA((2,)),
        ],
        compiler_params=pltpu.CompilerParams(
            dimension_semantics=("arbitrary",),
        ),
    )(weight.astype(jnp.bfloat16), features, adj)


# speedup vs baseline: 1.0485x; 1.0032x over previous
import jax
import jax.numpy as jnp
from jax.experimental import pallas as pl
from jax.experimental.pallas import tpu as pltpu

_BI = 400
_CS = 2000


def _fused_body(w_ref, f_hbm, adj_ref, o_ref, s_ref, f_buf, sem):
    t = pl.program_id(0)
    n_rows = f_hbm.shape[0]

    @pl.when(t == 0)
    def _support():
        n_chunks = n_rows // _CS

        def chunk_copy(j):
            return pltpu.make_async_copy(
                f_hbm.at[pl.ds(j * _CS, _CS), :], f_buf.at[j % 2], sem.at[j % 2]
            )

        chunk_copy(0).start()
        for j in range(n_chunks):
            if j + 1 < n_chunks:
                chunk_copy(j + 1).start()
            chunk_copy(j).wait()
            s_ref[j * _CS:(j + 1) * _CS, :] = jnp.dot(
                f_buf[j % 2].astype(jnp.bfloat16),
                w_ref[...],
                preferred_element_type=jnp.float32,
            ).astype(jnp.bfloat16)

    o_ref[...] = jnp.maximum(
        jnp.dot(
            adj_ref[...].astype(jnp.bfloat16),
            s_ref[...],
            preferred_element_type=jnp.float32,
        ),
        0.0,
    )


def kernel(features, adj, weight):
    n, d_in = features.shape
    d_out = weight.shape[1]

    return pl.pallas_call(
        _fused_body,
        grid=(n // _BI,),
        in_specs=[
            pl.BlockSpec((d_in, d_out), lambda i: (0, 0)),
            pl.BlockSpec(memory_space=pl.ANY),
            pl.BlockSpec((_BI, n), lambda i: (i, 0)),
        ],
        out_specs=pl.BlockSpec((_BI, d_out), lambda i: (i, 0)),
        out_shape=jax.ShapeDtypeStruct((n, d_out), jnp.float32),
        scratch_shapes=[
            pltpu.VMEM((n, d_out), jnp.bfloat16),
            pltpu.VMEM((2, _CS, d_in), jnp.float32),
            pltpu.SemaphoreType.DMA((2,)),
        ],
        compiler_params=pltpu.CompilerParams(
            dimension_semantics=("arbitrary",),
        ),
    )(weight.astype(jnp.bfloat16), features, adj)
